# baseline (speedup 1.0000x reference)
import functools

import jax
import jax.numpy as jnp
from jax import lax
from jax.experimental import pallas as pl
from jax.experimental.pallas import tpu as pltpu
from jax.experimental.pallas import tpu_sc as plsc

N = 100000
E = 1600000
H = 32

BN = 1024
N_PAD = 100352
HALF = 50176
ROWS_PER_TILE = HALF // 16
CHUNK = 256
BLOCK = 1024
N_BLOCKS = 1605632 // 16 // 1024
E_PAD = 1605632
EPT = E_PAD // 16
N_CHUNKS = EPT // CHUNK
NEG = -1e30


def _encode_body(aa_ref, at_ref, aaemb_ref, atemb_ref, pw1_ref, pb1_ref,
                 pw2_ref, pb2_ref, o_ref, ol_ref, ca_ref):
    aa = aa_ref[...]
    at = at_ref[...]
    aa_oh = (aa == lax.broadcasted_iota(jnp.int32, (1, 21), 1)).astype(jnp.float32)
    at_oh = (at == lax.broadcasted_iota(jnp.int32, (1, 3), 1)).astype(jnp.float32)
    e1 = jnp.dot(aa_oh, aaemb_ref[...], preferred_element_type=jnp.float32)
    e2 = jnp.dot(at_oh, atemb_ref[...], preferred_element_type=jnp.float32)
    h = jnp.concatenate([e1, e2], axis=-1)
    h = jnp.maximum(jnp.dot(h, pw1_ref[...], preferred_element_type=jnp.float32)
                    + pb1_ref[...], 0.0)
    h = jnp.dot(h, pw2_ref[...], preferred_element_type=jnp.float32) + pb2_ref[...]
    ca = (at == 1).astype(jnp.float32)
    o_ref[...] = h
    ol_ref[...] = jnp.where(ca > 0.0, h, NEG)
    ca_ref[...] = ca


def _encode(aa_idx, atom_idx, params):
    aa2 = aa_idx.reshape(N_PAD, 1)
    at2 = atom_idx.reshape(N_PAD, 1)
    return pl.pallas_call(
        _encode_body,
        grid=(N_PAD // BN,),
        in_specs=[
            pl.BlockSpec((BN, 1), lambda i: (i, 0)),
            pl.BlockSpec((BN, 1), lambda i: (i, 0)),
            pl.BlockSpec((21, 16), lambda i: (0, 0)),
            pl.BlockSpec((3, 16), lambda i: (0, 0)),
            pl.BlockSpec((32, H), lambda i: (0, 0)),
            pl.BlockSpec((H,), lambda i: (0,)),
            pl.BlockSpec((H, H), lambda i: (0, 0)),
            pl.BlockSpec((H,), lambda i: (0,)),
        ],
        out_specs=[
            pl.BlockSpec((BN, H), lambda i: (i, 0)),
            pl.BlockSpec((BN, H), lambda i: (i, 0)),
            pl.BlockSpec((BN, 1), lambda i: (i, 0)),
        ],
        out_shape=[
            jax.ShapeDtypeStruct((N_PAD, H), jnp.float32),
            jax.ShapeDtypeStruct((N_PAD, H), jnp.float32),
            jax.ShapeDtypeStruct((N_PAD, 1), jnp.float32),
        ],
    )(aa2, at2, params['aa_emb'], params['atom_emb'], params['proj_w1'],
      params['proj_b1'], params['proj_w2'], params['proj_b2'])


def _post_body(with_head, h_ref, ags_ref, agl_ref, ca_ref, es_ref, el_ref,
               sw1_ref, sb1_ref, sw2_ref, sb2_ref,
               lw1_ref, lb1_ref, lw2_ref, lb2_ref,
               g_ref, b_ref, *rest):
    h = h_ref[...]
    outs = es_ref[0, 0] * h + ags_ref[...]
    hs = jnp.dot(
        jnp.maximum(jnp.dot(outs, sw1_ref[...], preferred_element_type=jnp.float32)
                    + sb1_ref[...], 0.0),
        sw2_ref[...], preferred_element_type=jnp.float32) + sb2_ref[...]
    ca = ca_ref[...]
    outl = el_ref[0, 0] * h + ca * agl_ref[...]
    hl = jnp.dot(
        jnp.maximum(jnp.dot(outl, lw1_ref[...], preferred_element_type=jnp.float32)
                    + lb1_ref[...], 0.0),
        lw2_ref[...], preferred_element_type=jnp.float32) + lb2_ref[...]
    t = h + hs + hl
    mu = jnp.mean(t, axis=-1, keepdims=True)
    var = jnp.mean((t - mu) * (t - mu), axis=-1, keepdims=True)
    t = (t - mu) * lax.rsqrt(var + 1e-5) * g_ref[...] + b_ref[...]
    t = jnp.maximum(t, 0.0)
    if with_head:
        hw1_ref, hb1_ref, hw2_ref, hb2_ref, hw3_ref, hb3_ref, o_ref = rest
        t = jnp.maximum(jnp.dot(t, hw1_ref[...], preferred_element_type=jnp.float32)
                        + hb1_ref[...], 0.0)
        t = jnp.maximum(jnp.dot(t, hw2_ref[...], preferred_element_type=jnp.float32)
                        + hb2_ref[...], 0.0)
        o_ref[...] = jnp.dot(t, hw3_ref[...], preferred_element_type=jnp.float32) + hb3_ref[...]
    else:
        o_ref, ol_ref = rest
        o_ref[...] = t
        ol_ref[...] = jnp.where(ca > 0.0, t, NEG)


def _post(h, ags, agl, ca, lp, params, with_head):
    sp, lo = lp['short'], lp['long']
    es = jnp.reshape(1.0 + sp['eps'], (1, 1)).astype(jnp.float32)
    el = jnp.reshape(1.0 + lo['eps'], (1, 1)).astype(jnp.float32)
    mat = lambda r, c: pl.BlockSpec((r, c), lambda i: (0, 0))
    vec = lambda n: pl.BlockSpec((n,), lambda i: (0,))
    blk = lambda d: pl.BlockSpec((BN, d), lambda i: (i, 0))
    in_specs = [
        blk(H), blk(H), blk(H), blk(1),
        mat(1, 1), mat(1, 1),
        mat(H, H), vec(H), mat(H, H), vec(H),
        mat(H, H), vec(H), mat(H, H), vec(H),
        vec(H), vec(H),
    ]
    args = [h, ags, agl, ca, es, el,
            sp['nn_w1'], sp['nn_b1'], sp['nn_w2'], sp['nn_b2'],
            lo['nn_w1'], lo['nn_b1'], lo['nn_w2'], lo['nn_b2'],
            lp['ln_g'], lp['ln_b']]
    if with_head:
        in_specs += [mat(H, 16), vec(16), mat(16, 8), vec(8), mat(8, 8), vec(8)]
        args += [params['head_w1'], params['head_b1'], params['head_w2'],
                 params['head_b2'], params['head_w3'], params['head_b3']]
        out_specs = blk(8)
        out_shape = jax.ShapeDtypeStruct((N_PAD, 8), jnp.float32)
    else:
        out_specs = [blk(H), blk(H)]
        out_shape = [jax.ShapeDtypeStruct((N_PAD, H), jnp.float32),
                     jax.ShapeDtypeStruct((N_PAD, H), jnp.float32)]
    return pl.pallas_call(
        functools.partial(_post_body, with_head),
        grid=(N_PAD // BN,),
        in_specs=in_specs,
        out_specs=out_specs,
        out_shape=out_shape,
    )(*args)


def _conv_sc(x, srcp, dstp, eat, w, b, thresh, inclusive):
    mesh = plsc.VectorSubcoreMesh(core_axis_name="c", subcore_axis_name="s")

    @functools.partial(
        pl.kernel, mesh=mesh,
        out_type=jax.ShapeDtypeStruct((N_PAD, H), jnp.float32),
        compiler_params=pltpu.CompilerParams(use_tc_tiling_on_sc=False),
        scratch_types=[
            pltpu.VMEM((4, H), jnp.float32),
            pltpu.VMEM((H,), jnp.float32),
            pltpu.VMEM((BLOCK,), jnp.int32),
            pltpu.VMEM((BLOCK,), jnp.int32),
            pltpu.VMEM((BLOCK,), jnp.int32),
            pltpu.VMEM((BLOCK,), jnp.int32),
            pltpu.VMEM((4, BLOCK), jnp.float32),
            pltpu.VMEM((4, BLOCK), jnp.float32),
            pltpu.VMEM((128, H), jnp.float32),
            pltpu.VMEM((128, H), jnp.float32),
            pltpu.VMEM((BLOCK // 128, 128), jnp.int32),
            pltpu.VMEM_SHARED((HALF + 16, H), jnp.float32),
            pltpu.SemaphoreType.DMA,
            pltpu.SemaphoreType.DMA,
            pltpu.SemaphoreType.DMA,
            pltpu.SemaphoreType.DMA,
            pltpu.SemaphoreType.DMA,
            pltpu.SemaphoreType.DMA,
        ],
    )
    def body(x_hbm, src_hbm, dst_hbm, eat_hbm, w_hbm, b_hbm, agg_hbm,
             wv, bv, srcA, srcB, dstA, dstB, eaA, eaB, xr0, xr1,
             dl, slab, semLA, semLB, semG0, semG1, semS0, semS1):
        c = lax.axis_index("c")
        s = lax.axis_index("s")
        pltpu.sync_copy(w_hbm, wv)
        pltpu.sync_copy(b_hbm, bv)
        wlo = [wv[k, pl.ds(0, 16)] for k in range(4)]
        whi = [wv[k, pl.ds(16, 16)] for k in range(4)]
        blo = bv[pl.ds(0, 16)]
        bhi = bv[pl.ds(16, 16)]
        xrs = [xr0, xr1]
        gsems = [semG0, semG1]
        ssems = [semS0, semS1]
        iota16 = lax.iota(jnp.int32, 16)

        def zrow(r, _):
            xr0[r, pl.ds(0, 16)] = jnp.zeros((16,), jnp.float32)
            xr0[r, pl.ds(16, 16)] = jnp.zeros((16,), jnp.float32)
            return 0

        lax.fori_loop(0, 128, zrow, 0)
        zb = pl.multiple_of(s * ROWS_PER_TILE, 64)
        nfull = ROWS_PER_TILE // 128
        rem = ROWS_PER_TILE % 128
        for k in range(nfull):
            pltpu.sync_copy(xr0, slab.at[pl.ds(zb + k * 128, 128)])
        if rem:
            pltpu.sync_copy(xr0.at[pl.ds(0, rem)],
                            slab.at[pl.ds(zb + nfull * 128, rem)])
        plsc.subcore_barrier()

        ebase = pl.multiple_of(s * EPT, BLOCK)

        def fire_rec(bi, srcb, dstb, eab, sem):
            ci = jnp.where(bi < N_BLOCKS, bi, 0)
            cb = pl.multiple_of(ebase + ci * BLOCK, BLOCK)
            pltpu.async_copy(src_hbm.at[pl.ds(cb, BLOCK)], srcb, sem)
            pltpu.async_copy(dst_hbm.at[pl.ds(cb, BLOCK)], dstb, sem)
            for k in range(4):
                pltpu.async_copy(eat_hbm.at[k, pl.ds(cb, BLOCK)],
                                 eab.at[k], sem)

        def wait_rec(srcb, dstb, eab, sem):
            pltpu.make_async_copy(src_hbm.at[pl.ds(0, BLOCK)], srcb, sem).wait()
            pltpu.make_async_copy(dst_hbm.at[pl.ds(0, BLOCK)], dstb, sem).wait()
            for k in range(4):
                pltpu.make_async_copy(eat_hbm.at[k, pl.ds(0, BLOCK)],
                                      eab.at[k], sem).wait()

        def prepass(dstb, eab):
            def grp(g, _):
                off = pl.multiple_of(g * 16, 16)
                d = dstb[pl.ds(off, 16)]
                dist = eab[0, pl.ds(off, 16)]
                ge0 = jnp.where(dist >= 0.0, 1.0, 0.0)
                if inclusive:
                    thr = jnp.where(dist <= thresh, 1.0, 0.0)
                else:
                    thr = jnp.where(dist < thresh, 1.0, 0.0)
                hi = jnp.where(d >= HALF, 1, 0)
                loc = d - hi * HALF
                side = jnp.where(hi == c, 1.0, 0.0)
                predf = ge0 * thr * side
                loc = jnp.where(predf > 0.5, loc, HALF + iota16)
                dl[g // 8, pl.ds((g % 8) * 16, 16)] = loc
                return 0

            lax.fori_loop(0, BLOCK // 16, grp, 0)

        def wait_scatter(p, q):
            pltpu.make_async_copy(xrs[p], slab.at[dl.at[q]], ssems[p]).wait()

        def process_block(srcb, dstb, eab, semL, first):
            wait_rec(srcb, dstb, eab, semL)

            prepass(dstb, eab)
            pltpu.async_copy(x_hbm.at[srcb.at[pl.ds(0, 128)]], xr0, semG0)
            for q in range(BLOCK // 128):
                xrb = xrs[q % 2]
                pltpu.make_async_copy(x_hbm.at[pl.ds(0, 128)], xrb,
                                      gsems[q % 2]).wait()
                base = q * 128

                def egrp(g, _):
                    off = pl.multiple_of(base + g * 16, 16)
                    av = [eab[k, pl.ds(off, 16)] for k in range(4)]
                    for k in range(16):
                        e = g * 16 + k
                        a = [jnp.broadcast_to(av[t][k], (16,)) for t in range(4)]
                        elo = blo + a[0] * wlo[0] + a[1] * wlo[1] + a[2] * wlo[2] + a[3] * wlo[3]
                        ehi = bhi + a[0] * whi[0] + a[1] * whi[1] + a[2] * whi[2] + a[3] * whi[3]
                        xlo = xrb[e, pl.ds(0, 16)]
                        xhi = xrb[e, pl.ds(16, 16)]
                        xrb[e, pl.ds(0, 16)] = jnp.maximum(xlo + elo, 0.0)
                        xrb[e, pl.ds(16, 16)] = jnp.maximum(xhi + ehi, 0.0)
                    return 0

                lax.fori_loop(0, 8, egrp, 0)
                if q < BLOCK // 128 - 1:
                    p = (q + 1) % 2
                    o = (q + 1) * 128
                    pltpu.async_copy(x_hbm.at[srcb.at[pl.ds(o, 128)]],
                                     xrs[p], gsems[p])
                pltpu.sync_copy(xrb, slab.at[dl.at[q]], add=True)

        fire_rec(jnp.int32(0), srcA, dstA, eaA, semLA)
        fire_rec(jnp.int32(1), srcB, dstB, eaB, semLB)

        def pipe(j, _):
            b2 = j * 2
            process_block(srcA, dstA, eaA, semLA, j == 0)
            fire_rec(b2 + 2, srcA, dstA, eaA, semLA)
            process_block(srcB, dstB, eaB, semLB, jnp.bool_(False))
            fire_rec(b2 + 3, srcB, dstB, eaB, semLB)
            return 0

        lax.fori_loop(0, N_BLOCKS // 2, pipe, 0)
        wait_rec(srcA, dstA, eaA, semLA)
        wait_rec(srcB, dstB, eaB, semLB)
        plsc.subcore_barrier()
        rb = pl.multiple_of(s * ROWS_PER_TILE, 64)
        ob = pl.multiple_of(c * HALF + s * ROWS_PER_TILE, 64)
        pltpu.sync_copy(slab.at[pl.ds(rb, ROWS_PER_TILE)],
                        agg_hbm.at[pl.ds(ob, ROWS_PER_TILE)])

    return body(x, srcp, dstp, eat, w, b)


def kernel(aa_idx, atom_idx, edge_index, edge_attr, params):
    aa_p = jnp.pad(aa_idx.astype(jnp.int32), (0, N_PAD - N))
    at_p = jnp.pad(atom_idx.astype(jnp.int32), (0, N_PAD - N))
    src_p = jnp.pad(edge_index[0].astype(jnp.int32), (0, E_PAD - E))
    dst_p = jnp.pad(edge_index[1].astype(jnp.int32), (0, E_PAD - E))
    eat = jnp.pad(edge_attr.astype(jnp.float32).T, ((0, 0), (0, E_PAD - E)),
                  constant_values=-1.0)

    h, hlong, ca = _encode(aa_p, at_p, params)

    for li, lp in enumerate(params['layers']):
        ags = _conv_sc(h, src_p, dst_p, eat, lp['short']['lin_w'],
                       lp['short']['lin_b'], 10.0, False)
        agl = _conv_sc(hlong, src_p, dst_p, eat, lp['long']['lin_w'],
                       lp['long']['lin_b'], 25.0, True)
        with_head = li == len(params['layers']) - 1
        if with_head:
            out = _post(h, ags, agl, ca, lp, params, True)
        else:
            h, hlong = _post(h, ags, agl, ca, lp, params, False)
    return out[:N]


# speedup vs baseline: 1.3089x; 1.3089x over previous
import functools

import jax
import jax.numpy as jnp
from jax import lax
from jax.experimental import pallas as pl
from jax.experimental.pallas import tpu as pltpu
from jax.experimental.pallas import tpu_sc as plsc

N = 100000
E = 1600000
H = 32

BN = 1024
N_PAD = 100352
HALF = 50176
ROWS_PER_TILE = HALF // 16
CHUNK = 256
BLOCK = 1024
N_BLOCKS = 1605632 // 16 // 1024
E_PAD = 1605632
EPT = E_PAD // 16
N_CHUNKS = EPT // CHUNK
NEG = -1e30


def _encode_body(aa_ref, at_ref, aaemb_ref, atemb_ref, pw1_ref, pb1_ref,
                 pw2_ref, pb2_ref, o_ref, ol_ref, ca_ref):
    aa = aa_ref[...]
    at = at_ref[...]
    aa_oh = (aa == lax.broadcasted_iota(jnp.int32, (1, 21), 1)).astype(jnp.float32)
    at_oh = (at == lax.broadcasted_iota(jnp.int32, (1, 3), 1)).astype(jnp.float32)
    e1 = jnp.dot(aa_oh, aaemb_ref[...], preferred_element_type=jnp.float32)
    e2 = jnp.dot(at_oh, atemb_ref[...], preferred_element_type=jnp.float32)
    h = jnp.concatenate([e1, e2], axis=-1)
    h = jnp.maximum(jnp.dot(h, pw1_ref[...], preferred_element_type=jnp.float32)
                    + pb1_ref[...], 0.0)
    h = jnp.dot(h, pw2_ref[...], preferred_element_type=jnp.float32) + pb2_ref[...]
    ca = (at == 1).astype(jnp.float32)
    o_ref[...] = h
    ol_ref[...] = jnp.where(ca > 0.0, h, NEG)
    ca_ref[...] = ca


def _encode(aa_idx, atom_idx, params):
    aa2 = aa_idx.reshape(N_PAD, 1)
    at2 = atom_idx.reshape(N_PAD, 1)
    return pl.pallas_call(
        _encode_body,
        grid=(N_PAD // BN,),
        in_specs=[
            pl.BlockSpec((BN, 1), lambda i: (i, 0)),
            pl.BlockSpec((BN, 1), lambda i: (i, 0)),
            pl.BlockSpec((21, 16), lambda i: (0, 0)),
            pl.BlockSpec((3, 16), lambda i: (0, 0)),
            pl.BlockSpec((32, H), lambda i: (0, 0)),
            pl.BlockSpec((H,), lambda i: (0,)),
            pl.BlockSpec((H, H), lambda i: (0, 0)),
            pl.BlockSpec((H,), lambda i: (0,)),
        ],
        out_specs=[
            pl.BlockSpec((BN, H), lambda i: (i, 0)),
            pl.BlockSpec((BN, H), lambda i: (i, 0)),
            pl.BlockSpec((BN, 1), lambda i: (i, 0)),
        ],
        out_shape=[
            jax.ShapeDtypeStruct((N_PAD, H), jnp.float32),
            jax.ShapeDtypeStruct((N_PAD, H), jnp.float32),
            jax.ShapeDtypeStruct((N_PAD, 1), jnp.float32),
        ],
    )(aa2, at2, params['aa_emb'], params['atom_emb'], params['proj_w1'],
      params['proj_b1'], params['proj_w2'], params['proj_b2'])


def _post_body(with_head, h_ref, ags_ref, agl_ref, ca_ref, es_ref, el_ref,
               sw1_ref, sb1_ref, sw2_ref, sb2_ref,
               lw1_ref, lb1_ref, lw2_ref, lb2_ref,
               g_ref, b_ref, *rest):
    h = h_ref[...]
    outs = es_ref[0, 0] * h + ags_ref[...]
    hs = jnp.dot(
        jnp.maximum(jnp.dot(outs, sw1_ref[...], preferred_element_type=jnp.float32)
                    + sb1_ref[...], 0.0),
        sw2_ref[...], preferred_element_type=jnp.float32) + sb2_ref[...]
    ca = ca_ref[...]
    outl = el_ref[0, 0] * h + ca * agl_ref[...]
    hl = jnp.dot(
        jnp.maximum(jnp.dot(outl, lw1_ref[...], preferred_element_type=jnp.float32)
                    + lb1_ref[...], 0.0),
        lw2_ref[...], preferred_element_type=jnp.float32) + lb2_ref[...]
    t = h + hs + hl
    mu = jnp.mean(t, axis=-1, keepdims=True)
    var = jnp.mean((t - mu) * (t - mu), axis=-1, keepdims=True)
    t = (t - mu) * lax.rsqrt(var + 1e-5) * g_ref[...] + b_ref[...]
    t = jnp.maximum(t, 0.0)
    if with_head:
        hw1_ref, hb1_ref, hw2_ref, hb2_ref, hw3_ref, hb3_ref, o_ref = rest
        t = jnp.maximum(jnp.dot(t, hw1_ref[...], preferred_element_type=jnp.float32)
                        + hb1_ref[...], 0.0)
        t = jnp.maximum(jnp.dot(t, hw2_ref[...], preferred_element_type=jnp.float32)
                        + hb2_ref[...], 0.0)
        o_ref[...] = jnp.dot(t, hw3_ref[...], preferred_element_type=jnp.float32) + hb3_ref[...]
    else:
        o_ref, ol_ref = rest
        o_ref[...] = t
        ol_ref[...] = jnp.where(ca > 0.0, t, NEG)


def _post(h, ags, agl, ca, lp, params, with_head):
    sp, lo = lp['short'], lp['long']
    es = jnp.reshape(1.0 + sp['eps'], (1, 1)).astype(jnp.float32)
    el = jnp.reshape(1.0 + lo['eps'], (1, 1)).astype(jnp.float32)
    mat = lambda r, c: pl.BlockSpec((r, c), lambda i: (0, 0))
    vec = lambda n: pl.BlockSpec((n,), lambda i: (0,))
    blk = lambda d: pl.BlockSpec((BN, d), lambda i: (i, 0))
    in_specs = [
        blk(H), blk(H), blk(H), blk(1),
        mat(1, 1), mat(1, 1),
        mat(H, H), vec(H), mat(H, H), vec(H),
        mat(H, H), vec(H), mat(H, H), vec(H),
        vec(H), vec(H),
    ]
    args = [h, ags, agl, ca, es, el,
            sp['nn_w1'], sp['nn_b1'], sp['nn_w2'], sp['nn_b2'],
            lo['nn_w1'], lo['nn_b1'], lo['nn_w2'], lo['nn_b2'],
            lp['ln_g'], lp['ln_b']]
    if with_head:
        in_specs += [mat(H, 16), vec(16), mat(16, 8), vec(8), mat(8, 8), vec(8)]
        args += [params['head_w1'], params['head_b1'], params['head_w2'],
                 params['head_b2'], params['head_w3'], params['head_b3']]
        out_specs = blk(8)
        out_shape = jax.ShapeDtypeStruct((N_PAD, 8), jnp.float32)
    else:
        out_specs = [blk(H), blk(H)]
        out_shape = [jax.ShapeDtypeStruct((N_PAD, H), jnp.float32),
                     jax.ShapeDtypeStruct((N_PAD, H), jnp.float32)]
    return pl.pallas_call(
        functools.partial(_post_body, with_head),
        grid=(N_PAD // BN,),
        in_specs=in_specs,
        out_specs=out_specs,
        out_shape=out_shape,
    )(*args)


def _conv_sc(x, srcp, dstp, eat, w, b, thresh, inclusive):
    mesh = plsc.VectorSubcoreMesh(core_axis_name="c", subcore_axis_name="s")

    @functools.partial(
        pl.kernel, mesh=mesh,
        out_type=jax.ShapeDtypeStruct((N_PAD, H), jnp.float32),
        compiler_params=pltpu.CompilerParams(use_tc_tiling_on_sc=False),
        scratch_types=[
            pltpu.VMEM((4, H), jnp.float32),
            pltpu.VMEM((H,), jnp.float32),
            pltpu.VMEM((BLOCK,), jnp.int32),
            pltpu.VMEM((BLOCK,), jnp.int32),
            pltpu.VMEM((BLOCK,), jnp.int32),
            pltpu.VMEM((BLOCK,), jnp.int32),
            pltpu.VMEM((4, BLOCK), jnp.float32),
            pltpu.VMEM((4, BLOCK), jnp.float32),
            pltpu.VMEM((128, H), jnp.float32),
            pltpu.VMEM((128, H), jnp.float32),
            pltpu.VMEM((BLOCK // 128, 128), jnp.int32),
            pltpu.VMEM_SHARED((HALF + 16, H), jnp.float32),
            pltpu.SemaphoreType.DMA,
            pltpu.SemaphoreType.DMA,
            pltpu.SemaphoreType.DMA,
            pltpu.SemaphoreType.DMA,
            pltpu.SemaphoreType.DMA,
            pltpu.SemaphoreType.DMA,
        ],
    )
    def body(x_hbm, src_hbm, dst_hbm, eat_hbm, w_hbm, b_hbm, agg_hbm,
             wv, bv, srcA, srcB, dstA, dstB, eaA, eaB, xr0, xr1,
             dl, slab, semLA, semLB, semG0, semG1, semS0, semS1):
        c = lax.axis_index("c")
        s = lax.axis_index("s")
        pltpu.sync_copy(w_hbm, wv)
        pltpu.sync_copy(b_hbm, bv)
        wlo = [wv[k, pl.ds(0, 16)] for k in range(4)]
        whi = [wv[k, pl.ds(16, 16)] for k in range(4)]
        blo = bv[pl.ds(0, 16)]
        bhi = bv[pl.ds(16, 16)]
        xrs = [xr0, xr1]
        gsems = [semG0, semG1]
        ssems = [semS0, semS1]
        iota16 = lax.iota(jnp.int32, 16)

        def zrow(r, _):
            xr0[r, pl.ds(0, 16)] = jnp.zeros((16,), jnp.float32)
            xr0[r, pl.ds(16, 16)] = jnp.zeros((16,), jnp.float32)
            return 0

        lax.fori_loop(0, 128, zrow, 0)
        zb = pl.multiple_of(s * ROWS_PER_TILE, 64)
        nfull = ROWS_PER_TILE // 128
        rem = ROWS_PER_TILE % 128
        for k in range(nfull):
            pltpu.sync_copy(xr0, slab.at[pl.ds(zb + k * 128, 128)])
        if rem:
            pltpu.sync_copy(xr0.at[pl.ds(0, rem)],
                            slab.at[pl.ds(zb + nfull * 128, rem)])
        plsc.subcore_barrier()

        ebase = pl.multiple_of(s * EPT, BLOCK)

        def fire_rec(bi, srcb, dstb, eab, sem):
            ci = jnp.where(bi < N_BLOCKS, bi, 0)
            cb = pl.multiple_of(ebase + ci * BLOCK, BLOCK)
            pltpu.async_copy(src_hbm.at[pl.ds(cb, BLOCK)], srcb, sem)
            pltpu.async_copy(dst_hbm.at[pl.ds(cb, BLOCK)], dstb, sem)
            for k in range(4):
                pltpu.async_copy(eat_hbm.at[k, pl.ds(cb, BLOCK)],
                                 eab.at[k], sem)

        def wait_rec(srcb, dstb, eab, sem):
            pltpu.make_async_copy(src_hbm.at[pl.ds(0, BLOCK)], srcb, sem).wait()
            pltpu.make_async_copy(dst_hbm.at[pl.ds(0, BLOCK)], dstb, sem).wait()
            for k in range(4):
                pltpu.make_async_copy(eat_hbm.at[k, pl.ds(0, BLOCK)],
                                      eab.at[k], sem).wait()

        def prepass(dstb, eab):
            def grp(g, _):
                off = pl.multiple_of(g * 16, 16)
                d = dstb[pl.ds(off, 16)]
                dist = eab[0, pl.ds(off, 16)]
                ge0 = jnp.where(dist >= 0.0, 1.0, 0.0)
                if inclusive:
                    thr = jnp.where(dist <= thresh, 1.0, 0.0)
                else:
                    thr = jnp.where(dist < thresh, 1.0, 0.0)
                hi = jnp.where(d >= HALF, 1, 0)
                loc = d - hi * HALF
                side = jnp.where(hi == c, 1.0, 0.0)
                predf = ge0 * thr * side
                loc = jnp.where(predf > 0.5, loc, HALF + iota16)
                dl[g // 8, pl.ds((g % 8) * 16, 16)] = loc
                return 0

            lax.fori_loop(0, BLOCK // 16, grp, 0)

        def wait_scatter(p, q):
            pltpu.make_async_copy(xrs[p], slab.at[dl.at[q]], ssems[p]).wait()

        def process_block(srcb, dstb, eab, semL, first):
            wait_rec(srcb, dstb, eab, semL)

            prepass(dstb, eab)
            pltpu.async_copy(x_hbm.at[srcb.at[pl.ds(0, 128)]], xr0, semG0)
            for q in range(BLOCK // 128):
                if q < BLOCK // 128 - 1:
                    o = (q + 1) * 128
                    pltpu.async_copy(x_hbm.at[srcb.at[pl.ds(o, 128)]],
                                     xrs[(q + 1) % 2], gsems[(q + 1) % 2])
                xrb = xrs[q % 2]
                pltpu.make_async_copy(x_hbm.at[pl.ds(0, 128)], xrb,
                                      gsems[q % 2]).wait()
                base = q * 128

                def egrp(g, _):
                    off = pl.multiple_of(base + g * 16, 16)
                    av = [eab[k, pl.ds(off, 16)] for k in range(4)]
                    for k in range(16):
                        e = g * 16 + k
                        a = [jnp.broadcast_to(av[t][k], (16,)) for t in range(4)]
                        elo = blo + a[0] * wlo[0] + a[1] * wlo[1] + a[2] * wlo[2] + a[3] * wlo[3]
                        ehi = bhi + a[0] * whi[0] + a[1] * whi[1] + a[2] * whi[2] + a[3] * whi[3]
                        xlo = xrb[e, pl.ds(0, 16)]
                        xhi = xrb[e, pl.ds(16, 16)]
                        xrb[e, pl.ds(0, 16)] = jnp.maximum(xlo + elo, 0.0)
                        xrb[e, pl.ds(16, 16)] = jnp.maximum(xhi + ehi, 0.0)
                    return 0

                lax.fori_loop(0, 8, egrp, 0)
                pltpu.sync_copy(xrb, slab.at[dl.at[q]], add=True)

        fire_rec(jnp.int32(0), srcA, dstA, eaA, semLA)
        fire_rec(jnp.int32(1), srcB, dstB, eaB, semLB)

        def pipe(j, _):
            b2 = j * 2
            process_block(srcA, dstA, eaA, semLA, j == 0)
            fire_rec(b2 + 2, srcA, dstA, eaA, semLA)
            process_block(srcB, dstB, eaB, semLB, jnp.bool_(False))
            fire_rec(b2 + 3, srcB, dstB, eaB, semLB)
            return 0

        lax.fori_loop(0, N_BLOCKS // 2, pipe, 0)
        wait_rec(srcA, dstA, eaA, semLA)
        wait_rec(srcB, dstB, eaB, semLB)
        plsc.subcore_barrier()
        rb = pl.multiple_of(s * ROWS_PER_TILE, 64)
        ob = pl.multiple_of(c * HALF + s * ROWS_PER_TILE, 64)
        pltpu.sync_copy(slab.at[pl.ds(rb, ROWS_PER_TILE)],
                        agg_hbm.at[pl.ds(ob, ROWS_PER_TILE)])

    return body(x, srcp, dstp, eat, w, b)


def kernel(aa_idx, atom_idx, edge_index, edge_attr, params):
    aa_p = jnp.pad(aa_idx.astype(jnp.int32), (0, N_PAD - N))
    at_p = jnp.pad(atom_idx.astype(jnp.int32), (0, N_PAD - N))
    src_p = jnp.pad(edge_index[0].astype(jnp.int32), (0, E_PAD - E))
    dst_p = jnp.pad(edge_index[1].astype(jnp.int32), (0, E_PAD - E))
    eat = jnp.pad(edge_attr.astype(jnp.float32).T, ((0, 0), (0, E_PAD - E)),
                  constant_values=-1.0)

    h, hlong, ca = _encode(aa_p, at_p, params)

    for li, lp in enumerate(params['layers']):
        ags = _conv_sc(h, src_p, dst_p, eat, lp['short']['lin_w'],
                       lp['short']['lin_b'], 10.0, False)
        agl = _conv_sc(hlong, src_p, dst_p, eat, lp['long']['lin_w'],
                       lp['long']['lin_b'], 25.0, True)
        with_head = li == len(params['layers']) - 1
        if with_head:
            out = _post(h, ags, agl, ca, lp, params, True)
        else:
            h, hlong = _post(h, ags, agl, ca, lp, params, False)
    return out[:N]


# speedup vs baseline: 1.3359x; 1.0206x over previous
import functools

import jax
import jax.numpy as jnp
from jax import lax
from jax.experimental import pallas as pl
from jax.experimental.pallas import tpu as pltpu
from jax.experimental.pallas import tpu_sc as plsc

N = 100000
E = 1600000
H = 32

BN = 1024
N_PAD = 100352
HALF = 50176
ROWS_PER_TILE = HALF // 16
CHUNK = 256
BLOCK = 1024
N_BLOCKS = 1605632 // 16 // 1024
E_PAD = 1605632
EPT = E_PAD // 16
N_CHUNKS = EPT // CHUNK
NEG = -1e30


def _encode_body(aa_ref, at_ref, aaemb_ref, atemb_ref, pw1_ref, pb1_ref,
                 pw2_ref, pb2_ref, o_ref, ol_ref, ca_ref):
    aa = aa_ref[...]
    at = at_ref[...]
    aa_oh = (aa == lax.broadcasted_iota(jnp.int32, (1, 21), 1)).astype(jnp.float32)
    at_oh = (at == lax.broadcasted_iota(jnp.int32, (1, 3), 1)).astype(jnp.float32)
    e1 = jnp.dot(aa_oh, aaemb_ref[...], preferred_element_type=jnp.float32)
    e2 = jnp.dot(at_oh, atemb_ref[...], preferred_element_type=jnp.float32)
    h = jnp.concatenate([e1, e2], axis=-1)
    h = jnp.maximum(jnp.dot(h, pw1_ref[...], preferred_element_type=jnp.float32)
                    + pb1_ref[...], 0.0)
    h = jnp.dot(h, pw2_ref[...], preferred_element_type=jnp.float32) + pb2_ref[...]
    ca = (at == 1).astype(jnp.float32)
    o_ref[...] = h
    ol_ref[...] = jnp.where(ca > 0.0, h, NEG)
    ca_ref[...] = ca


def _encode(aa_idx, atom_idx, params):
    aa2 = aa_idx.reshape(N_PAD, 1)
    at2 = atom_idx.reshape(N_PAD, 1)
    return pl.pallas_call(
        _encode_body,
        grid=(N_PAD // BN,),
        in_specs=[
            pl.BlockSpec((BN, 1), lambda i: (i, 0)),
            pl.BlockSpec((BN, 1), lambda i: (i, 0)),
            pl.BlockSpec((21, 16), lambda i: (0, 0)),
            pl.BlockSpec((3, 16), lambda i: (0, 0)),
            pl.BlockSpec((32, H), lambda i: (0, 0)),
            pl.BlockSpec((H,), lambda i: (0,)),
            pl.BlockSpec((H, H), lambda i: (0, 0)),
            pl.BlockSpec((H,), lambda i: (0,)),
        ],
        out_specs=[
            pl.BlockSpec((BN, H), lambda i: (i, 0)),
            pl.BlockSpec((BN, H), lambda i: (i, 0)),
            pl.BlockSpec((BN, 1), lambda i: (i, 0)),
        ],
        out_shape=[
            jax.ShapeDtypeStruct((N_PAD, H), jnp.float32),
            jax.ShapeDtypeStruct((N_PAD, H), jnp.float32),
            jax.ShapeDtypeStruct((N_PAD, 1), jnp.float32),
        ],
    )(aa2, at2, params['aa_emb'], params['atom_emb'], params['proj_w1'],
      params['proj_b1'], params['proj_w2'], params['proj_b2'])


def _post_body(with_head, h_ref, ags_ref, agl_ref, ca_ref, es_ref, el_ref,
               sw1_ref, sb1_ref, sw2_ref, sb2_ref,
               lw1_ref, lb1_ref, lw2_ref, lb2_ref,
               g_ref, b_ref, *rest):
    h = h_ref[...]
    outs = es_ref[0, 0] * h + ags_ref[...]
    hs = jnp.dot(
        jnp.maximum(jnp.dot(outs, sw1_ref[...], preferred_element_type=jnp.float32)
                    + sb1_ref[...], 0.0),
        sw2_ref[...], preferred_element_type=jnp.float32) + sb2_ref[...]
    ca = ca_ref[...]
    outl = el_ref[0, 0] * h + ca * agl_ref[...]
    hl = jnp.dot(
        jnp.maximum(jnp.dot(outl, lw1_ref[...], preferred_element_type=jnp.float32)
                    + lb1_ref[...], 0.0),
        lw2_ref[...], preferred_element_type=jnp.float32) + lb2_ref[...]
    t = h + hs + hl
    mu = jnp.mean(t, axis=-1, keepdims=True)
    var = jnp.mean((t - mu) * (t - mu), axis=-1, keepdims=True)
    t = (t - mu) * lax.rsqrt(var + 1e-5) * g_ref[...] + b_ref[...]
    t = jnp.maximum(t, 0.0)
    if with_head:
        hw1_ref, hb1_ref, hw2_ref, hb2_ref, hw3_ref, hb3_ref, o_ref = rest
        t = jnp.maximum(jnp.dot(t, hw1_ref[...], preferred_element_type=jnp.float32)
                        + hb1_ref[...], 0.0)
        t = jnp.maximum(jnp.dot(t, hw2_ref[...], preferred_element_type=jnp.float32)
                        + hb2_ref[...], 0.0)
        o_ref[...] = jnp.dot(t, hw3_ref[...], preferred_element_type=jnp.float32) + hb3_ref[...]
    else:
        o_ref, ol_ref = rest
        o_ref[...] = t
        ol_ref[...] = jnp.where(ca > 0.0, t, NEG)


def _post(h, ags, agl, ca, lp, params, with_head):
    sp, lo = lp['short'], lp['long']
    es = jnp.reshape(1.0 + sp['eps'], (1, 1)).astype(jnp.float32)
    el = jnp.reshape(1.0 + lo['eps'], (1, 1)).astype(jnp.float32)
    mat = lambda r, c: pl.BlockSpec((r, c), lambda i: (0, 0))
    vec = lambda n: pl.BlockSpec((n,), lambda i: (0,))
    blk = lambda d: pl.BlockSpec((BN, d), lambda i: (i, 0))
    in_specs = [
        blk(H), blk(H), blk(H), blk(1),
        mat(1, 1), mat(1, 1),
        mat(H, H), vec(H), mat(H, H), vec(H),
        mat(H, H), vec(H), mat(H, H), vec(H),
        vec(H), vec(H),
    ]
    args = [h, ags, agl, ca, es, el,
            sp['nn_w1'], sp['nn_b1'], sp['nn_w2'], sp['nn_b2'],
            lo['nn_w1'], lo['nn_b1'], lo['nn_w2'], lo['nn_b2'],
            lp['ln_g'], lp['ln_b']]
    if with_head:
        in_specs += [mat(H, 16), vec(16), mat(16, 8), vec(8), mat(8, 8), vec(8)]
        args += [params['head_w1'], params['head_b1'], params['head_w2'],
                 params['head_b2'], params['head_w3'], params['head_b3']]
        out_specs = blk(8)
        out_shape = jax.ShapeDtypeStruct((N_PAD, 8), jnp.float32)
    else:
        out_specs = [blk(H), blk(H)]
        out_shape = [jax.ShapeDtypeStruct((N_PAD, H), jnp.float32),
                     jax.ShapeDtypeStruct((N_PAD, H), jnp.float32)]
    return pl.pallas_call(
        functools.partial(_post_body, with_head),
        grid=(N_PAD // BN,),
        in_specs=in_specs,
        out_specs=out_specs,
        out_shape=out_shape,
    )(*args)


def _conv_sc(x, srcp, dstp, eat, w, b, thresh, inclusive):
    mesh = plsc.VectorSubcoreMesh(core_axis_name="c", subcore_axis_name="s")

    @functools.partial(
        pl.kernel, mesh=mesh,
        out_type=jax.ShapeDtypeStruct((N_PAD, H), jnp.float32),
        compiler_params=pltpu.CompilerParams(use_tc_tiling_on_sc=False),
        scratch_types=[
            pltpu.VMEM((4, H), jnp.float32),
            pltpu.VMEM((H,), jnp.float32),
            pltpu.VMEM((BLOCK,), jnp.int32),
            pltpu.VMEM((BLOCK,), jnp.int32),
            pltpu.VMEM((BLOCK,), jnp.int32),
            pltpu.VMEM((BLOCK,), jnp.int32),
            pltpu.VMEM((4, BLOCK), jnp.float32),
            pltpu.VMEM((4, BLOCK), jnp.float32),
            pltpu.VMEM((128, H), jnp.float32),
            pltpu.VMEM((128, H), jnp.float32),
            pltpu.VMEM((BLOCK // 128, 128), jnp.int32),
            pltpu.VMEM_SHARED((HALF + 16, H), jnp.float32),
            pltpu.SemaphoreType.DMA,
            pltpu.SemaphoreType.DMA,
            pltpu.SemaphoreType.DMA,
            pltpu.SemaphoreType.DMA,
            pltpu.SemaphoreType.DMA,
            pltpu.SemaphoreType.DMA,
        ],
    )
    def body(x_hbm, src_hbm, dst_hbm, eat_hbm, w_hbm, b_hbm, agg_hbm,
             wv, bv, srcA, srcB, dstA, dstB, eaA, eaB, xr0, xr1,
             dl, slab, semLA, semLB, semG0, semG1, semS0, semS1):
        c = lax.axis_index("c")
        s = lax.axis_index("s")
        pltpu.sync_copy(w_hbm, wv)
        pltpu.sync_copy(b_hbm, bv)
        wlo = [wv[k, pl.ds(0, 16)] for k in range(4)]
        whi = [wv[k, pl.ds(16, 16)] for k in range(4)]
        blo = bv[pl.ds(0, 16)]
        bhi = bv[pl.ds(16, 16)]
        xrs = [xr0, xr1]
        gsems = [semG0, semG1]
        ssems = [semS0, semS1]
        iota16 = lax.iota(jnp.int32, 16)

        def zrow(r, _):
            xr0[r, pl.ds(0, 16)] = jnp.zeros((16,), jnp.float32)
            xr0[r, pl.ds(16, 16)] = jnp.zeros((16,), jnp.float32)
            return 0

        lax.fori_loop(0, 128, zrow, 0)
        zb = pl.multiple_of(s * ROWS_PER_TILE, 64)
        nfull = ROWS_PER_TILE // 128
        rem = ROWS_PER_TILE % 128
        for k in range(nfull):
            pltpu.sync_copy(xr0, slab.at[pl.ds(zb + k * 128, 128)])
        if rem:
            pltpu.sync_copy(xr0.at[pl.ds(0, rem)],
                            slab.at[pl.ds(zb + nfull * 128, rem)])
        plsc.subcore_barrier()

        ebase = pl.multiple_of(s * EPT, BLOCK)

        def fire_rec(bi, srcb, dstb, eab, sem):
            ci = jnp.where(bi < N_BLOCKS, bi, 0)
            cb = pl.multiple_of(ebase + ci * BLOCK, BLOCK)
            pltpu.async_copy(src_hbm.at[pl.ds(cb, BLOCK)], srcb, sem)
            pltpu.async_copy(dst_hbm.at[pl.ds(cb, BLOCK)], dstb, sem)
            for k in range(4):
                pltpu.async_copy(eat_hbm.at[k, pl.ds(cb, BLOCK)],
                                 eab.at[k], sem)

        def wait_rec(srcb, dstb, eab, sem):
            pltpu.make_async_copy(src_hbm.at[pl.ds(0, BLOCK)], srcb, sem).wait()
            pltpu.make_async_copy(dst_hbm.at[pl.ds(0, BLOCK)], dstb, sem).wait()
            for k in range(4):
                pltpu.make_async_copy(eat_hbm.at[k, pl.ds(0, BLOCK)],
                                      eab.at[k], sem).wait()

        def prepass(dstb, eab):
            def grp(g, _):
                off = pl.multiple_of(g * 16, 16)
                d = dstb[pl.ds(off, 16)]
                dist = eab[0, pl.ds(off, 16)]
                ge0 = jnp.where(dist >= 0.0, 1.0, 0.0)
                if inclusive:
                    thr = jnp.where(dist <= thresh, 1.0, 0.0)
                else:
                    thr = jnp.where(dist < thresh, 1.0, 0.0)
                hi = jnp.where(d >= HALF, 1, 0)
                loc = d - hi * HALF
                side = jnp.where(hi == c, 1.0, 0.0)
                predf = ge0 * thr * side
                loc = jnp.where(predf > 0.5, loc, HALF + iota16)
                dl[g // 8, pl.ds((g % 8) * 16, 16)] = loc
                return 0

            lax.fori_loop(0, BLOCK // 16, grp, 0)

        def wait_scatter(p, q):
            pltpu.make_async_copy(xrs[p], slab.at[dl.at[q]], ssems[p]).wait()

        def process_block(srcb, dstb, eab, semL, first):
            wait_rec(srcb, dstb, eab, semL)

            pltpu.async_copy(x_hbm.at[srcb.at[pl.ds(0, 128)]], xr0, semG0)
            pltpu.async_copy(x_hbm.at[srcb.at[pl.ds(128, 128)]], xr1, semG1)
            prepass(dstb, eab)
            for q in range(BLOCK // 128):
                xrb = xrs[q % 2]
                pltpu.make_async_copy(x_hbm.at[pl.ds(0, 128)], xrb,
                                      gsems[q % 2]).wait()
                base = q * 128

                def egrp(g, _):
                    off = pl.multiple_of(base + g * 16, 16)
                    av = [eab[k, pl.ds(off, 16)] for k in range(4)]
                    for k in range(16):
                        e = g * 16 + k
                        a = [jnp.broadcast_to(av[t][k], (16,)) for t in range(4)]
                        elo = blo + a[0] * wlo[0] + a[1] * wlo[1] + a[2] * wlo[2] + a[3] * wlo[3]
                        ehi = bhi + a[0] * whi[0] + a[1] * whi[1] + a[2] * whi[2] + a[3] * whi[3]
                        xlo = xrb[e, pl.ds(0, 16)]
                        xhi = xrb[e, pl.ds(16, 16)]
                        xrb[e, pl.ds(0, 16)] = jnp.maximum(xlo + elo, 0.0)
                        xrb[e, pl.ds(16, 16)] = jnp.maximum(xhi + ehi, 0.0)
                    return 0

                lax.fori_loop(0, 8, egrp, 0)
                pltpu.sync_copy(xrb, slab.at[dl.at[q]], add=True)
                if q + 2 < BLOCK // 128:
                    o = (q + 2) * 128
                    pltpu.async_copy(x_hbm.at[srcb.at[pl.ds(o, 128)]],
                                     xrb, gsems[q % 2])

        fire_rec(jnp.int32(0), srcA, dstA, eaA, semLA)
        fire_rec(jnp.int32(1), srcB, dstB, eaB, semLB)

        def pipe(j, _):
            b2 = j * 2
            process_block(srcA, dstA, eaA, semLA, j == 0)
            fire_rec(b2 + 2, srcA, dstA, eaA, semLA)
            process_block(srcB, dstB, eaB, semLB, jnp.bool_(False))
            fire_rec(b2 + 3, srcB, dstB, eaB, semLB)
            return 0

        lax.fori_loop(0, N_BLOCKS // 2, pipe, 0)
        wait_rec(srcA, dstA, eaA, semLA)
        wait_rec(srcB, dstB, eaB, semLB)
        plsc.subcore_barrier()
        rb = pl.multiple_of(s * ROWS_PER_TILE, 64)
        ob = pl.multiple_of(c * HALF + s * ROWS_PER_TILE, 64)
        pltpu.sync_copy(slab.at[pl.ds(rb, ROWS_PER_TILE)],
                        agg_hbm.at[pl.ds(ob, ROWS_PER_TILE)])

    return body(x, srcp, dstp, eat, w, b)


def kernel(aa_idx, atom_idx, edge_index, edge_attr, params):
    aa_p = jnp.pad(aa_idx.astype(jnp.int32), (0, N_PAD - N))
    at_p = jnp.pad(atom_idx.astype(jnp.int32), (0, N_PAD - N))
    src_p = jnp.pad(edge_index[0].astype(jnp.int32), (0, E_PAD - E))
    dst_p = jnp.pad(edge_index[1].astype(jnp.int32), (0, E_PAD - E))
    eat = jnp.pad(edge_attr.astype(jnp.float32).T, ((0, 0), (0, E_PAD - E)),
                  constant_values=-1.0)

    h, hlong, ca = _encode(aa_p, at_p, params)

    for li, lp in enumerate(params['layers']):
        ags = _conv_sc(h, src_p, dst_p, eat, lp['short']['lin_w'],
                       lp['short']['lin_b'], 10.0, False)
        agl = _conv_sc(hlong, src_p, dst_p, eat, lp['long']['lin_w'],
                       lp['long']['lin_b'], 25.0, True)
        with_head = li == len(params['layers']) - 1
        if with_head:
            out = _post(h, ags, agl, ca, lp, params, True)
        else:
            h, hlong = _post(h, ags, agl, ca, lp, params, False)
    return out[:N]
